# baseline (speedup 1.0000x reference)
import functools

import jax
import jax.numpy as jnp
from jax import lax
from jax.experimental import pallas as pl
from jax.experimental.pallas import tpu as pltpu
from jax.experimental.pallas import tpu_sc as plsc

NC = 2
NS = 16
EB = 128
RB = 256

_MESH = plsc.VectorSubcoreMesh(
    core_axis_name="c", subcore_axis_name="s", num_cores=NC, num_subcores=NS)


def _wid(c, s):
    return s * NC + c


def _deg_kernel(npad, ep):
    epw = ep // (NC * NS)
    nblk = epw // EB
    rpt = npad // NS

    def body(dst_hbm, zeros_hbm, e0_hbm, out_hbm, dst_v, e0_v, hist_sh, sem):
        c = lax.axis_index("c")
        s = lax.axis_index("s")
        row0 = pl.multiple_of(s * rpt, 8)
        pltpu.sync_copy(zeros_hbm.at[pl.ds(row0, rpt)],
                        hist_sh.at[pl.ds(row0, rpt)])
        pltpu.sync_copy(e0_hbm, e0_v)
        plsc.subcore_barrier()
        base = _wid(c, s) * epw

        def step(i, carry):
            off = pl.multiple_of(base + i * EB, EB)
            pltpu.sync_copy(dst_hbm.at[pl.ds(off, EB)], dst_v)
            pltpu.sync_copy(e0_v, hist_sh.at[dst_v], add=True)
            return carry

        lax.fori_loop(0, nblk, step, 0)
        plsc.subcore_barrier()
        pltpu.sync_copy(hist_sh.at[pl.ds(row0, rpt)],
                        out_hbm.at[c, pl.ds(row0, rpt)])

    return pl.kernel(
        body,
        out_type=jax.ShapeDtypeStruct((NC, npad, 16), jnp.float32),
        mesh=_MESH,
        compiler_params=pltpu.CompilerParams(use_tc_tiling_on_sc=False),
        scratch_types=[
            pltpu.VMEM((EB,), jnp.int32),
            pltpu.VMEM((EB, 16), jnp.float32),
            pltpu.VMEM_SHARED((npad, 16), jnp.float32),
            pltpu.SemaphoreType.DMA,
        ],
    )


def _row_scatter_kernel(npad, ep, f):
    epw = ep // (NC * NS)
    nblk = epw // EB
    rpt = npad // NS

    def body(hs1_hbm, src_hbm, dst_hbm, zeros_hbm, out_hbm,
             src_all, dst_a, dst_b, rows_a, rows_b, acc_sh,
             sem_a, sem_b, semd_a, semd_b):
        c = lax.axis_index("c")
        s = lax.axis_index("s")
        row0 = pl.multiple_of(s * rpt, 8)
        pltpu.sync_copy(zeros_hbm.at[pl.ds(row0, rpt)],
                        acc_sh.at[pl.ds(row0, rpt)])
        base = pl.multiple_of(_wid(c, s) * epw, EB)
        pltpu.sync_copy(src_hbm.at[pl.ds(base, epw)], src_all)
        plsc.subcore_barrier()

        def prefetch(i, buf, sem, dbuf, dsem):
            ioff = pl.multiple_of(i * EB, EB)
            off = pl.multiple_of(base + i * EB, EB)
            pltpu.async_copy(
                hs1_hbm.at[src_all.at[pl.ds(ioff, EB)]], buf, sem)
            pltpu.async_copy(dst_hbm.at[pl.ds(off, EB)], dbuf, dsem)

        def process(i, buf, sem, dbuf, dsem):
            ioff = pl.multiple_of(i * EB, EB)
            pltpu.make_async_copy(
                hs1_hbm.at[src_all.at[pl.ds(ioff, EB)]], buf, sem).wait()
            pltpu.make_async_copy(
                dst_hbm.at[pl.ds(base, EB)], dbuf, dsem).wait()
            pltpu.sync_copy(buf, acc_sh.at[dbuf], add=True)

        prefetch(0, rows_a, sem_a, dst_a, semd_a)

        def step(k, carry):
            i0 = k * 2
            i1 = i0 + 1

            @pl.when(i1 < nblk)
            def _ga():
                prefetch(i1, rows_b, sem_b, dst_b, semd_b)
            process(i0, rows_a, sem_a, dst_a, semd_a)

            @pl.when(i1 < nblk)
            def _pb():
                @pl.when(i1 + 1 < nblk)
                def _gb():
                    prefetch(i1 + 1, rows_a, sem_a, dst_a, semd_a)
                process(i1, rows_b, sem_b, dst_b, semd_b)
            return carry

        lax.fori_loop(0, (nblk + 1) // 2, step, 0)
        plsc.subcore_barrier()
        pltpu.sync_copy(acc_sh.at[pl.ds(row0, rpt)],
                        out_hbm.at[c, pl.ds(row0, rpt)])

    return pl.kernel(
        body,
        out_type=jax.ShapeDtypeStruct((NC, npad, f), jnp.float32),
        mesh=_MESH,
        scratch_types=[
            pltpu.VMEM((epw,), jnp.int32),
            pltpu.VMEM((EB,), jnp.int32),
            pltpu.VMEM((EB,), jnp.int32),
            pltpu.VMEM((EB, f), jnp.float32),
            pltpu.VMEM((EB, f), jnp.float32),
            pltpu.VMEM_SHARED((npad, f), jnp.float32),
            pltpu.SemaphoreType.DMA,
            pltpu.SemaphoreType.DMA,
            pltpu.SemaphoreType.DMA,
            pltpu.SemaphoreType.DMA,
        ],
    )


def _z_scatter_kernel(npad, ep):
    epw = ep // (NC * NS)
    nblk = epw // EB
    rpt = npad // NS

    def body(z_hbm, src_hbm, dst_hbm, zeros_hbm, out_hbm,
             src_all, dst_v, rows_v, z_sh, sz_sh, sem):
        c = lax.axis_index("c")
        s = lax.axis_index("s")
        row0 = pl.multiple_of(s * rpt, 8)
        pltpu.sync_copy(zeros_hbm.at[pl.ds(row0, rpt)],
                        sz_sh.at[pl.ds(row0, rpt)])
        pltpu.sync_copy(z_hbm.at[pl.ds(row0, rpt)],
                        z_sh.at[pl.ds(row0, rpt)])
        base = pl.multiple_of(_wid(c, s) * epw, EB)
        pltpu.sync_copy(src_hbm.at[pl.ds(base, epw)], src_all)
        plsc.subcore_barrier()

        def step(i, carry):
            ioff = pl.multiple_of(i * EB, EB)
            off = pl.multiple_of(base + i * EB, EB)
            pltpu.sync_copy(dst_hbm.at[pl.ds(off, EB)], dst_v)
            pltpu.async_copy(
                z_sh.at[src_all.at[pl.ds(ioff, EB)]], rows_v, sem).wait()
            pltpu.sync_copy(rows_v, sz_sh.at[dst_v], add=True)
            return carry

        lax.fori_loop(0, nblk, step, 0)
        plsc.subcore_barrier()
        pltpu.sync_copy(sz_sh.at[pl.ds(row0, rpt)],
                        out_hbm.at[c, pl.ds(row0, rpt)])

    return pl.kernel(
        body,
        out_type=jax.ShapeDtypeStruct((NC, npad, 16), jnp.float32),
        mesh=_MESH,
        compiler_params=pltpu.CompilerParams(use_tc_tiling_on_sc=False),
        scratch_types=[
            pltpu.VMEM((epw,), jnp.int32),
            pltpu.VMEM((EB,), jnp.int32),
            pltpu.VMEM((EB, 16), jnp.float32),
            pltpu.VMEM_SHARED((npad, 16), jnp.float32),
            pltpu.VMEM_SHARED((npad, 16), jnp.float32),
            pltpu.SemaphoreType.DMA,
        ],
    )


def _r16(a):
    return a.astype(jnp.bfloat16).astype(jnp.float32)


def _b_body(x_ref, w1_ref, degp_ref, hs1_ref, dinv_ref):
    s1 = degp_ref[0] + degp_ref[1]
    deg2 = jnp.sum(s1, axis=1, keepdims=True) + 1.0
    dinv2 = lax.rsqrt(deg2)
    h = jnp.dot(_r16(x_ref[...]), _r16(w1_ref[...]),
                preferred_element_type=jnp.float32,
                precision=lax.Precision.HIGHEST)
    hs1_ref[...] = h * dinv2
    dinv_ref[...] = dinv2


def _d_body(accp_ref, hs1_ref, dinv_ref, b1_ref, w2_ref, wm_ref, z_ref):
    dinv2 = dinv_ref[...]
    a = accp_ref[0] + accp_ref[1] + hs1_ref[...]
    h1 = a * dinv2 + b1_ref[...][None, :]
    h1r = jnp.maximum(h1, 0.0)
    wm2 = jnp.dot(_r16(w2_ref[...]), _r16(wm_ref[...]),
                  preferred_element_type=jnp.float32,
                  precision=lax.Precision.HIGHEST)
    zz = jnp.dot(_r16(h1r), wm2, preferred_element_type=jnp.float32,
                 precision=lax.Precision.HIGHEST) * dinv2
    lane = lax.broadcasted_iota(jnp.int32, (RB, 16), 1)
    z_ref[...] = jnp.where(lane == 0, jnp.broadcast_to(zz, (RB, 16)), 0.0)


def _f_body(g, z_ref, szp_ref, dinv_ref, batch_ref, b2_ref, wm_ref, bm_ref,
            out_ref, sums_scr, cnts_scr):
    i = pl.program_id(0)
    c22 = jnp.dot(b2_ref[...][None, :], wm_ref[...],
                  preferred_element_type=jnp.float32, precision=lax.Precision.HIGHEST)
    szs = szp_ref[0] + szp_ref[1]
    sz2 = jnp.sum(szs, axis=1, keepdims=True)
    zb2 = jnp.sum(z_ref[...], axis=1, keepdims=True)
    v2 = dinv_ref[...] * (sz2 + zb2) + c22
    onehot = (batch_ref[...] ==
              lax.broadcasted_iota(jnp.int32, (RB, g), 1)).astype(jnp.float32)
    ps = jnp.sum(onehot * v2, axis=0, keepdims=True)
    pc = jnp.sum(onehot, axis=0, keepdims=True)

    @pl.when(i == 0)
    def _init():
        sums_scr[...] = ps
        cnts_scr[...] = pc

    @pl.when(i > 0)
    def _accum():
        sums_scr[...] = sums_scr[...] + ps
        cnts_scr[...] = cnts_scr[...] + pc

    @pl.when(i == pl.num_programs(0) - 1)
    def _fin():
        pooled = sums_scr[...] / jnp.maximum(cnts_scr[...], 1.0)
        out_ref[...] = pooled + bm_ref[...][None, :]


def kernel(x, edge_index, batch, W1, b1, W2, b2, Wm, bm):
    n, f = x.shape
    h = W1.shape[1]
    e = edge_index.shape[1]
    g = 64

    npad = ((n + 2047) // 2048) * 2048
    ep = ((e + 32 * EB - 1) // (32 * EB)) * 32 * EB
    nb = npad // RB

    src = edge_index[0]
    dst = edge_index[1]
    pad_idx = (n + jnp.arange(ep - e, dtype=jnp.int32) % (npad - n)).astype(
        jnp.int32)
    src_p = jnp.concatenate([src.astype(jnp.int32), pad_idx])
    dst_p = jnp.concatenate([dst.astype(jnp.int32), pad_idx])
    x_p = jnp.pad(x, ((0, npad - n), (0, 0)))
    batch_p = jnp.pad(batch.astype(jnp.int32), (0, npad - n),
                      constant_values=g)[:, None]
    zeros16 = jnp.zeros((npad, 16), jnp.float32)
    zeros_f = jnp.zeros((npad, f), jnp.float32)
    e0rows = jnp.zeros((EB, 16), jnp.float32).at[:, 0].set(1.0)

    degp = _deg_kernel(npad, ep)(dst_p, zeros16, e0rows)

    hs1, dinv2d = pl.pallas_call(
        _b_body,
        grid=(nb,),
        in_specs=[
            pl.BlockSpec((RB, f), lambda i: (i, 0)),
            pl.BlockSpec((f, h), lambda i: (0, 0)),
            pl.BlockSpec((NC, RB, 16), lambda i: (0, i, 0)),
        ],
        out_specs=[
            pl.BlockSpec((RB, h), lambda i: (i, 0)),
            pl.BlockSpec((RB, 1), lambda i: (i, 0)),
        ],
        out_shape=[
            jax.ShapeDtypeStruct((npad, h), jnp.float32),
            jax.ShapeDtypeStruct((npad, 1), jnp.float32),
        ],
    )(x_p, W1, degp)

    accp = _row_scatter_kernel(npad, ep, h)(hs1, src_p, dst_p, zeros_f)

    z16 = pl.pallas_call(
        _d_body,
        grid=(nb,),
        in_specs=[
            pl.BlockSpec((NC, RB, h), lambda i: (0, i, 0)),
            pl.BlockSpec((RB, h), lambda i: (i, 0)),
            pl.BlockSpec((RB, 1), lambda i: (i, 0)),
            pl.BlockSpec((h,), lambda i: (0,)),
            pl.BlockSpec((h, h), lambda i: (0, 0)),
            pl.BlockSpec((h, 1), lambda i: (0, 0)),
        ],
        out_specs=pl.BlockSpec((RB, 16), lambda i: (i, 0)),
        out_shape=jax.ShapeDtypeStruct((npad, 16), jnp.float32),
    )(accp, hs1, dinv2d, b1, W2, Wm)

    szp = _z_scatter_kernel(npad, ep)(z16, src_p, dst_p, zeros16)

    out = pl.pallas_call(
        functools.partial(_f_body, g),
        grid=(nb,),
        in_specs=[
            pl.BlockSpec((RB, 16), lambda i: (i, 0)),
            pl.BlockSpec((NC, RB, 16), lambda i: (0, i, 0)),
            pl.BlockSpec((RB, 1), lambda i: (i, 0)),
            pl.BlockSpec((RB, 1), lambda i: (i, 0)),
            pl.BlockSpec((h,), lambda i: (0,)),
            pl.BlockSpec((h, 1), lambda i: (0, 0)),
            pl.BlockSpec((1,), lambda i: (0,)),
        ],
        out_specs=pl.BlockSpec((1, g), lambda i: (0, 0)),
        out_shape=jax.ShapeDtypeStruct((1, g), jnp.float32),
        scratch_shapes=[
            pltpu.VMEM((1, g), jnp.float32),
            pltpu.VMEM((1, g), jnp.float32),
        ],
        compiler_params=pltpu.CompilerParams(
            dimension_semantics=("arbitrary",)),
    )(z16, szp, dinv2d, batch_p, b2, Wm, bm)

    return out.reshape(g, 1)


# speedup vs baseline: 1.1864x; 1.1864x over previous
import functools

import jax
import jax.numpy as jnp
from jax import lax
from jax.experimental import pallas as pl
from jax.experimental.pallas import tpu as pltpu
from jax.experimental.pallas import tpu_sc as plsc

NC = 2
NS = 16
EB = 128
RB = 256

_MESH = plsc.VectorSubcoreMesh(
    core_axis_name="c", subcore_axis_name="s", num_cores=NC, num_subcores=NS)


def _wid(c, s):
    return s * NC + c


def _deg_kernel(npad, ep):
    epw = ep // (NC * NS)
    nblk = epw // EB
    rpt = npad // NS

    def body(dst_hbm, zeros_hbm, e0_hbm, out_hbm, dst_a, dst_b, e0_v,
             hist_sh, semd_a, semd_b):
        c = lax.axis_index("c")
        s = lax.axis_index("s")
        row0 = pl.multiple_of(s * rpt, 8)
        pltpu.sync_copy(zeros_hbm.at[pl.ds(row0, rpt)],
                        hist_sh.at[pl.ds(row0, rpt)])
        pltpu.sync_copy(e0_hbm, e0_v)
        plsc.subcore_barrier()
        base = pl.multiple_of(_wid(c, s) * epw, EB)

        def prefetch(i, dbuf, dsem):
            off = pl.multiple_of(base + i * EB, EB)
            pltpu.async_copy(dst_hbm.at[pl.ds(off, EB)], dbuf, dsem)

        def process(i, dbuf, dsem):
            pltpu.make_async_copy(
                dst_hbm.at[pl.ds(base, EB)], dbuf, dsem).wait()
            pltpu.sync_copy(e0_v, hist_sh.at[dbuf], add=True)

        prefetch(0, dst_a, semd_a)

        def step(k, carry):
            i0 = k * 2
            i1 = i0 + 1

            @pl.when(i1 < nblk)
            def _ga():
                prefetch(i1, dst_b, semd_b)
            process(i0, dst_a, semd_a)

            @pl.when(i1 < nblk)
            def _pb():
                @pl.when(i1 + 1 < nblk)
                def _gb():
                    prefetch(i1 + 1, dst_a, semd_a)
                process(i1, dst_b, semd_b)
            return carry

        lax.fori_loop(0, (nblk + 1) // 2, step, 0)
        plsc.subcore_barrier()
        pltpu.sync_copy(hist_sh.at[pl.ds(row0, rpt)],
                        out_hbm.at[c, pl.ds(row0, rpt)])

    return pl.kernel(
        body,
        out_type=jax.ShapeDtypeStruct((NC, npad, 16), jnp.float32),
        mesh=_MESH,
        compiler_params=pltpu.CompilerParams(use_tc_tiling_on_sc=False),
        scratch_types=[
            pltpu.VMEM((EB,), jnp.int32),
            pltpu.VMEM((EB,), jnp.int32),
            pltpu.VMEM((EB, 16), jnp.float32),
            pltpu.VMEM_SHARED((npad, 16), jnp.float32),
            pltpu.SemaphoreType.DMA,
            pltpu.SemaphoreType.DMA,
        ],
    )


def _row_scatter_kernel(npad, ep, f):
    epw = ep // (NC * NS)
    nblk = epw // EB
    rpt = npad // NS

    def body(hs1_hbm, src_hbm, dst_hbm, zeros_hbm, out_hbm,
             src_all, dst_a, dst_b, rows_a, rows_b, acc_sh,
             sem_a, sem_b, semd_a, semd_b):
        c = lax.axis_index("c")
        s = lax.axis_index("s")
        row0 = pl.multiple_of(s * rpt, 8)
        pltpu.sync_copy(zeros_hbm.at[pl.ds(row0, rpt)],
                        acc_sh.at[pl.ds(row0, rpt)])
        base = pl.multiple_of(_wid(c, s) * epw, EB)
        pltpu.sync_copy(src_hbm.at[pl.ds(base, epw)], src_all)
        plsc.subcore_barrier()

        def prefetch(i, buf, sem, dbuf, dsem):
            ioff = pl.multiple_of(i * EB, EB)
            off = pl.multiple_of(base + i * EB, EB)
            pltpu.async_copy(
                hs1_hbm.at[src_all.at[pl.ds(ioff, EB)]], buf, sem)
            pltpu.async_copy(dst_hbm.at[pl.ds(off, EB)], dbuf, dsem)

        def process(i, buf, sem, dbuf, dsem):
            ioff = pl.multiple_of(i * EB, EB)
            pltpu.make_async_copy(
                hs1_hbm.at[src_all.at[pl.ds(ioff, EB)]], buf, sem).wait()
            pltpu.make_async_copy(
                dst_hbm.at[pl.ds(base, EB)], dbuf, dsem).wait()
            pltpu.sync_copy(buf, acc_sh.at[dbuf], add=True)

        prefetch(0, rows_a, sem_a, dst_a, semd_a)

        def step(k, carry):
            i0 = k * 2
            i1 = i0 + 1

            @pl.when(i1 < nblk)
            def _ga():
                prefetch(i1, rows_b, sem_b, dst_b, semd_b)
            process(i0, rows_a, sem_a, dst_a, semd_a)

            @pl.when(i1 < nblk)
            def _pb():
                @pl.when(i1 + 1 < nblk)
                def _gb():
                    prefetch(i1 + 1, rows_a, sem_a, dst_a, semd_a)
                process(i1, rows_b, sem_b, dst_b, semd_b)
            return carry

        lax.fori_loop(0, (nblk + 1) // 2, step, 0)
        plsc.subcore_barrier()
        pltpu.sync_copy(acc_sh.at[pl.ds(row0, rpt)],
                        out_hbm.at[c, pl.ds(row0, rpt)])

    return pl.kernel(
        body,
        out_type=jax.ShapeDtypeStruct((NC, npad, f), jnp.float32),
        mesh=_MESH,
        scratch_types=[
            pltpu.VMEM((epw,), jnp.int32),
            pltpu.VMEM((EB,), jnp.int32),
            pltpu.VMEM((EB,), jnp.int32),
            pltpu.VMEM((EB, f), jnp.float32),
            pltpu.VMEM((EB, f), jnp.float32),
            pltpu.VMEM_SHARED((npad, f), jnp.float32),
            pltpu.SemaphoreType.DMA,
            pltpu.SemaphoreType.DMA,
            pltpu.SemaphoreType.DMA,
            pltpu.SemaphoreType.DMA,
        ],
    )


def _z_scatter_kernel(npad, ep):
    epw = ep // (NC * NS)
    nblk = epw // EB
    rpt = npad // NS

    def body(z_hbm, src_hbm, dst_hbm, zeros_hbm, out_hbm,
             src_all, dst_a, dst_b, rows_a, rows_b, z_sh, sz_sh,
             sem_a, sem_b, semd_a, semd_b):
        c = lax.axis_index("c")
        s = lax.axis_index("s")
        row0 = pl.multiple_of(s * rpt, 8)
        pltpu.sync_copy(zeros_hbm.at[pl.ds(row0, rpt)],
                        sz_sh.at[pl.ds(row0, rpt)])
        pltpu.sync_copy(z_hbm.at[pl.ds(row0, rpt)],
                        z_sh.at[pl.ds(row0, rpt)])
        base = pl.multiple_of(_wid(c, s) * epw, EB)
        pltpu.sync_copy(src_hbm.at[pl.ds(base, epw)], src_all)
        plsc.subcore_barrier()

        def prefetch(i, buf, sem, dbuf, dsem):
            ioff = pl.multiple_of(i * EB, EB)
            off = pl.multiple_of(base + i * EB, EB)
            pltpu.async_copy(
                z_sh.at[src_all.at[pl.ds(ioff, EB)]], buf, sem)
            pltpu.async_copy(dst_hbm.at[pl.ds(off, EB)], dbuf, dsem)

        def process(i, buf, sem, dbuf, dsem):
            ioff = pl.multiple_of(i * EB, EB)
            pltpu.make_async_copy(
                z_sh.at[src_all.at[pl.ds(ioff, EB)]], buf, sem).wait()
            pltpu.make_async_copy(
                dst_hbm.at[pl.ds(base, EB)], dbuf, dsem).wait()
            pltpu.sync_copy(buf, sz_sh.at[dbuf], add=True)

        prefetch(0, rows_a, sem_a, dst_a, semd_a)

        def step(k, carry):
            i0 = k * 2
            i1 = i0 + 1

            @pl.when(i1 < nblk)
            def _ga():
                prefetch(i1, rows_b, sem_b, dst_b, semd_b)
            process(i0, rows_a, sem_a, dst_a, semd_a)

            @pl.when(i1 < nblk)
            def _pb():
                @pl.when(i1 + 1 < nblk)
                def _gb():
                    prefetch(i1 + 1, rows_a, sem_a, dst_a, semd_a)
                process(i1, rows_b, sem_b, dst_b, semd_b)
            return carry

        lax.fori_loop(0, (nblk + 1) // 2, step, 0)
        plsc.subcore_barrier()
        pltpu.sync_copy(sz_sh.at[pl.ds(row0, rpt)],
                        out_hbm.at[c, pl.ds(row0, rpt)])

    return pl.kernel(
        body,
        out_type=jax.ShapeDtypeStruct((NC, npad, 16), jnp.float32),
        mesh=_MESH,
        compiler_params=pltpu.CompilerParams(use_tc_tiling_on_sc=False),
        scratch_types=[
            pltpu.VMEM((epw,), jnp.int32),
            pltpu.VMEM((EB,), jnp.int32),
            pltpu.VMEM((EB,), jnp.int32),
            pltpu.VMEM((EB, 16), jnp.float32),
            pltpu.VMEM((EB, 16), jnp.float32),
            pltpu.VMEM_SHARED((npad, 16), jnp.float32),
            pltpu.VMEM_SHARED((npad, 16), jnp.float32),
            pltpu.SemaphoreType.DMA,
            pltpu.SemaphoreType.DMA,
            pltpu.SemaphoreType.DMA,
            pltpu.SemaphoreType.DMA,
        ],
    )


def _r16(a):
    return a.astype(jnp.bfloat16).astype(jnp.float32)


def _b_body(x_ref, w1_ref, degp_ref, hs1_ref, dinv_ref):
    s1 = degp_ref[0] + degp_ref[1]
    deg2 = jnp.sum(s1, axis=1, keepdims=True) + 1.0
    dinv2 = lax.rsqrt(deg2)
    h = jnp.dot(_r16(x_ref[...]), _r16(w1_ref[...]),
                preferred_element_type=jnp.float32,
                precision=lax.Precision.HIGHEST)
    hs1_ref[...] = h * dinv2
    dinv_ref[...] = dinv2


def _d_body(accp_ref, hs1_ref, dinv_ref, b1_ref, w2_ref, wm_ref, z_ref):
    dinv2 = dinv_ref[...]
    a = accp_ref[0] + accp_ref[1] + hs1_ref[...]
    h1 = a * dinv2 + b1_ref[...][None, :]
    h1r = jnp.maximum(h1, 0.0)
    wm2 = jnp.dot(_r16(w2_ref[...]), _r16(wm_ref[...]),
                  preferred_element_type=jnp.float32,
                  precision=lax.Precision.HIGHEST)
    zz = jnp.dot(_r16(h1r), wm2, preferred_element_type=jnp.float32,
                 precision=lax.Precision.HIGHEST) * dinv2
    lane = lax.broadcasted_iota(jnp.int32, (RB, 16), 1)
    z_ref[...] = jnp.where(lane == 0, jnp.broadcast_to(zz, (RB, 16)), 0.0)


def _f_body(g, z_ref, szp_ref, dinv_ref, batch_ref, b2_ref, wm_ref, bm_ref,
            out_ref, sums_scr, cnts_scr):
    i = pl.program_id(0)
    c22 = jnp.dot(b2_ref[...][None, :], wm_ref[...],
                  preferred_element_type=jnp.float32, precision=lax.Precision.HIGHEST)
    szs = szp_ref[0] + szp_ref[1]
    sz2 = jnp.sum(szs, axis=1, keepdims=True)
    zb2 = jnp.sum(z_ref[...], axis=1, keepdims=True)
    v2 = dinv_ref[...] * (sz2 + zb2) + c22
    onehot = (batch_ref[...] ==
              lax.broadcasted_iota(jnp.int32, (RB, g), 1)).astype(jnp.float32)
    ps = jnp.sum(onehot * v2, axis=0, keepdims=True)
    pc = jnp.sum(onehot, axis=0, keepdims=True)

    @pl.when(i == 0)
    def _init():
        sums_scr[...] = ps
        cnts_scr[...] = pc

    @pl.when(i > 0)
    def _accum():
        sums_scr[...] = sums_scr[...] + ps
        cnts_scr[...] = cnts_scr[...] + pc

    @pl.when(i == pl.num_programs(0) - 1)
    def _fin():
        pooled = sums_scr[...] / jnp.maximum(cnts_scr[...], 1.0)
        out_ref[...] = pooled + bm_ref[...][None, :]


def kernel(x, edge_index, batch, W1, b1, W2, b2, Wm, bm):
    n, f = x.shape
    h = W1.shape[1]
    e = edge_index.shape[1]
    g = 64

    npad = ((n + 2047) // 2048) * 2048
    ep = ((e + 32 * EB - 1) // (32 * EB)) * 32 * EB
    nb = npad // RB

    src = edge_index[0]
    dst = edge_index[1]
    pad_idx = (n + jnp.arange(ep - e, dtype=jnp.int32) % (npad - n)).astype(
        jnp.int32)
    src_p = jnp.concatenate([src.astype(jnp.int32), pad_idx])
    dst_p = jnp.concatenate([dst.astype(jnp.int32), pad_idx])
    x_p = jnp.pad(x, ((0, npad - n), (0, 0)))
    batch_p = jnp.pad(batch.astype(jnp.int32), (0, npad - n),
                      constant_values=g)[:, None]
    zeros16 = jnp.zeros((npad, 16), jnp.float32)
    zeros_f = jnp.zeros((npad, f), jnp.float32)
    e0rows = jnp.zeros((EB, 16), jnp.float32).at[:, 0].set(1.0)

    degp = _deg_kernel(npad, ep)(dst_p, zeros16, e0rows)

    hs1, dinv2d = pl.pallas_call(
        _b_body,
        grid=(nb,),
        in_specs=[
            pl.BlockSpec((RB, f), lambda i: (i, 0)),
            pl.BlockSpec((f, h), lambda i: (0, 0)),
            pl.BlockSpec((NC, RB, 16), lambda i: (0, i, 0)),
        ],
        out_specs=[
            pl.BlockSpec((RB, h), lambda i: (i, 0)),
            pl.BlockSpec((RB, 1), lambda i: (i, 0)),
        ],
        out_shape=[
            jax.ShapeDtypeStruct((npad, h), jnp.float32),
            jax.ShapeDtypeStruct((npad, 1), jnp.float32),
        ],
    )(x_p, W1, degp)

    accp = _row_scatter_kernel(npad, ep, h)(hs1, src_p, dst_p, zeros_f)

    z16 = pl.pallas_call(
        _d_body,
        grid=(nb,),
        in_specs=[
            pl.BlockSpec((NC, RB, h), lambda i: (0, i, 0)),
            pl.BlockSpec((RB, h), lambda i: (i, 0)),
            pl.BlockSpec((RB, 1), lambda i: (i, 0)),
            pl.BlockSpec((h,), lambda i: (0,)),
            pl.BlockSpec((h, h), lambda i: (0, 0)),
            pl.BlockSpec((h, 1), lambda i: (0, 0)),
        ],
        out_specs=pl.BlockSpec((RB, 16), lambda i: (i, 0)),
        out_shape=jax.ShapeDtypeStruct((npad, 16), jnp.float32),
    )(accp, hs1, dinv2d, b1, W2, Wm)

    szp = _z_scatter_kernel(npad, ep)(z16, src_p, dst_p, zeros16)

    out = pl.pallas_call(
        functools.partial(_f_body, g),
        grid=(nb,),
        in_specs=[
            pl.BlockSpec((RB, 16), lambda i: (i, 0)),
            pl.BlockSpec((NC, RB, 16), lambda i: (0, i, 0)),
            pl.BlockSpec((RB, 1), lambda i: (i, 0)),
            pl.BlockSpec((RB, 1), lambda i: (i, 0)),
            pl.BlockSpec((h,), lambda i: (0,)),
            pl.BlockSpec((h, 1), lambda i: (0, 0)),
            pl.BlockSpec((1,), lambda i: (0,)),
        ],
        out_specs=pl.BlockSpec((1, g), lambda i: (0, 0)),
        out_shape=jax.ShapeDtypeStruct((1, g), jnp.float32),
        scratch_shapes=[
            pltpu.VMEM((1, g), jnp.float32),
            pltpu.VMEM((1, g), jnp.float32),
        ],
        compiler_params=pltpu.CompilerParams(
            dimension_semantics=("arbitrary",)),
    )(z16, szp, dinv2d, batch_p, b2, Wm, bm)

    return out.reshape(g, 1)


# speedup vs baseline: 1.2716x; 1.0718x over previous
import functools

import jax
import jax.numpy as jnp
from jax import lax
from jax.experimental import pallas as pl
from jax.experimental.pallas import tpu as pltpu
from jax.experimental.pallas import tpu_sc as plsc

NC = 2
NS = 16
EB = 128
EBA = 2048
EBE = 1024
RB = 256

_MESH = plsc.VectorSubcoreMesh(
    core_axis_name="c", subcore_axis_name="s", num_cores=NC, num_subcores=NS)


def _wid(c, s):
    return s * NC + c


def _deg_kernel(npad, ep):
    epw = ep // (NC * NS)
    nblk = epw // EBA
    rpt = npad // NS

    def body(dst_hbm, zeros_hbm, e0_hbm, out_hbm, dst_a, dst_b, e0_v,
             hist_sh, semd_a, semd_b):
        c = lax.axis_index("c")
        s = lax.axis_index("s")
        row0 = pl.multiple_of(s * rpt, 8)
        pltpu.sync_copy(zeros_hbm.at[pl.ds(row0, rpt)],
                        hist_sh.at[pl.ds(row0, rpt)])
        pltpu.sync_copy(e0_hbm, e0_v)
        plsc.subcore_barrier()
        base = pl.multiple_of(_wid(c, s) * epw, EBA)

        def prefetch(i, dbuf, dsem):
            off = pl.multiple_of(base + i * EBA, EB)
            pltpu.async_copy(dst_hbm.at[pl.ds(off, EBA)], dbuf, dsem)

        def process(i, dbuf, dsem):
            pltpu.make_async_copy(
                dst_hbm.at[pl.ds(base, EBA)], dbuf, dsem).wait()
            pltpu.sync_copy(e0_v, hist_sh.at[dbuf], add=True)

        prefetch(0, dst_a, semd_a)

        def step(k, carry):
            i0 = k * 2
            i1 = i0 + 1

            @pl.when(i1 < nblk)
            def _ga():
                prefetch(i1, dst_b, semd_b)
            process(i0, dst_a, semd_a)

            @pl.when(i1 < nblk)
            def _pb():
                @pl.when(i1 + 1 < nblk)
                def _gb():
                    prefetch(i1 + 1, dst_a, semd_a)
                process(i1, dst_b, semd_b)
            return carry

        lax.fori_loop(0, (nblk + 1) // 2, step, 0)
        plsc.subcore_barrier()
        pltpu.sync_copy(hist_sh.at[pl.ds(row0, rpt)],
                        out_hbm.at[c, pl.ds(row0, rpt)])

    return pl.kernel(
        body,
        out_type=jax.ShapeDtypeStruct((NC, npad, 16), jnp.float32),
        mesh=_MESH,
        compiler_params=pltpu.CompilerParams(use_tc_tiling_on_sc=False),
        scratch_types=[
            pltpu.VMEM((EBA,), jnp.int32),
            pltpu.VMEM((EBA,), jnp.int32),
            pltpu.VMEM((EBA, 16), jnp.float32),
            pltpu.VMEM_SHARED((npad, 16), jnp.float32),
            pltpu.SemaphoreType.DMA,
            pltpu.SemaphoreType.DMA,
        ],
    )


def _row_scatter_kernel(npad, ep, f):
    epw = ep // (NC * NS)
    nblk = epw // EB
    rpt = npad // NS

    def body(hs1_hbm, src_hbm, dst_hbm, zeros_hbm, out_hbm,
             src_all, dst_a, dst_b, rows_a, rows_b, acc_sh,
             sem_a, sem_b, semd_a, semd_b):
        c = lax.axis_index("c")
        s = lax.axis_index("s")
        row0 = pl.multiple_of(s * rpt, 8)
        pltpu.sync_copy(zeros_hbm.at[pl.ds(row0, rpt)],
                        acc_sh.at[pl.ds(row0, rpt)])
        base = pl.multiple_of(_wid(c, s) * epw, EB)
        pltpu.sync_copy(src_hbm.at[pl.ds(base, epw)], src_all)
        plsc.subcore_barrier()

        def prefetch(i, buf, sem, dbuf, dsem):
            ioff = pl.multiple_of(i * EB, EB)
            off = pl.multiple_of(base + i * EB, EB)
            pltpu.async_copy(
                hs1_hbm.at[src_all.at[pl.ds(ioff, EB)]], buf, sem)
            pltpu.async_copy(dst_hbm.at[pl.ds(off, EB)], dbuf, dsem)

        def process(i, buf, sem, dbuf, dsem):
            ioff = pl.multiple_of(i * EB, EB)
            pltpu.make_async_copy(
                hs1_hbm.at[src_all.at[pl.ds(ioff, EB)]], buf, sem).wait()
            pltpu.make_async_copy(
                dst_hbm.at[pl.ds(base, EB)], dbuf, dsem).wait()
            pltpu.sync_copy(buf, acc_sh.at[dbuf], add=True)

        prefetch(0, rows_a, sem_a, dst_a, semd_a)

        def step(k, carry):
            i0 = k * 2
            i1 = i0 + 1

            @pl.when(i1 < nblk)
            def _ga():
                prefetch(i1, rows_b, sem_b, dst_b, semd_b)
            process(i0, rows_a, sem_a, dst_a, semd_a)

            @pl.when(i1 < nblk)
            def _pb():
                @pl.when(i1 + 1 < nblk)
                def _gb():
                    prefetch(i1 + 1, rows_a, sem_a, dst_a, semd_a)
                process(i1, rows_b, sem_b, dst_b, semd_b)
            return carry

        lax.fori_loop(0, (nblk + 1) // 2, step, 0)
        plsc.subcore_barrier()
        pltpu.sync_copy(acc_sh.at[pl.ds(row0, rpt)],
                        out_hbm.at[c, pl.ds(row0, rpt)])

    return pl.kernel(
        body,
        out_type=jax.ShapeDtypeStruct((NC, npad, f), jnp.float32),
        mesh=_MESH,
        scratch_types=[
            pltpu.VMEM((epw,), jnp.int32),
            pltpu.VMEM((EB,), jnp.int32),
            pltpu.VMEM((EB,), jnp.int32),
            pltpu.VMEM((EB, f), jnp.float32),
            pltpu.VMEM((EB, f), jnp.float32),
            pltpu.VMEM_SHARED((npad, f), jnp.float32),
            pltpu.SemaphoreType.DMA,
            pltpu.SemaphoreType.DMA,
            pltpu.SemaphoreType.DMA,
            pltpu.SemaphoreType.DMA,
        ],
    )


def _z_scatter_kernel(npad, ep):
    epw = ep // (NC * NS)
    nblk = epw // EBE
    rpt = npad // NS

    def body(z_hbm, src_hbm, dst_hbm, zeros_hbm, out_hbm,
             src_all, dst_a, dst_b, rows_a, rows_b, z_sh, sz_sh,
             sem_a, sem_b, semd_a, semd_b):
        c = lax.axis_index("c")
        s = lax.axis_index("s")
        row0 = pl.multiple_of(s * rpt, 8)
        pltpu.sync_copy(zeros_hbm.at[pl.ds(row0, rpt)],
                        sz_sh.at[pl.ds(row0, rpt)])
        pltpu.sync_copy(z_hbm.at[pl.ds(row0, rpt)],
                        z_sh.at[pl.ds(row0, rpt)])
        base = pl.multiple_of(_wid(c, s) * epw, EBE)
        pltpu.sync_copy(src_hbm.at[pl.ds(base, epw)], src_all)
        plsc.subcore_barrier()

        def prefetch(i, buf, sem, dbuf, dsem):
            ioff = pl.multiple_of(i * EBE, EBE)
            off = pl.multiple_of(base + i * EBE, EBE)
            pltpu.async_copy(
                z_sh.at[src_all.at[pl.ds(ioff, EBE)]], buf, sem)
            pltpu.async_copy(dst_hbm.at[pl.ds(off, EBE)], dbuf, dsem)

        def process(i, buf, sem, dbuf, dsem):
            ioff = pl.multiple_of(i * EBE, EBE)
            pltpu.make_async_copy(
                z_sh.at[src_all.at[pl.ds(ioff, EBE)]], buf, sem).wait()
            pltpu.make_async_copy(
                dst_hbm.at[pl.ds(base, EBE)], dbuf, dsem).wait()
            pltpu.sync_copy(buf, sz_sh.at[dbuf], add=True)

        prefetch(0, rows_a, sem_a, dst_a, semd_a)

        def step(k, carry):
            i0 = k * 2
            i1 = i0 + 1

            @pl.when(i1 < nblk)
            def _ga():
                prefetch(i1, rows_b, sem_b, dst_b, semd_b)
            process(i0, rows_a, sem_a, dst_a, semd_a)

            @pl.when(i1 < nblk)
            def _pb():
                @pl.when(i1 + 1 < nblk)
                def _gb():
                    prefetch(i1 + 1, rows_a, sem_a, dst_a, semd_a)
                process(i1, rows_b, sem_b, dst_b, semd_b)
            return carry

        lax.fori_loop(0, (nblk + 1) // 2, step, 0)
        plsc.subcore_barrier()
        pltpu.sync_copy(sz_sh.at[pl.ds(row0, rpt)],
                        out_hbm.at[c, pl.ds(row0, rpt)])

    return pl.kernel(
        body,
        out_type=jax.ShapeDtypeStruct((NC, npad, 16), jnp.float32),
        mesh=_MESH,
        compiler_params=pltpu.CompilerParams(use_tc_tiling_on_sc=False),
        scratch_types=[
            pltpu.VMEM((epw,), jnp.int32),
            pltpu.VMEM((EBE,), jnp.int32),
            pltpu.VMEM((EBE,), jnp.int32),
            pltpu.VMEM((EBE, 16), jnp.float32),
            pltpu.VMEM((EBE, 16), jnp.float32),
            pltpu.VMEM_SHARED((npad, 16), jnp.float32),
            pltpu.VMEM_SHARED((npad, 16), jnp.float32),
            pltpu.SemaphoreType.DMA,
            pltpu.SemaphoreType.DMA,
            pltpu.SemaphoreType.DMA,
            pltpu.SemaphoreType.DMA,
        ],
    )


def _r16(a):
    return a.astype(jnp.bfloat16).astype(jnp.float32)


def _b_body(x_ref, w1_ref, degp_ref, hs1_ref, dinv_ref):
    s1 = degp_ref[0] + degp_ref[1]
    deg2 = jnp.sum(s1, axis=1, keepdims=True) + 1.0
    dinv2 = lax.rsqrt(deg2)
    h = jnp.dot(_r16(x_ref[...]), _r16(w1_ref[...]),
                preferred_element_type=jnp.float32,
                precision=lax.Precision.HIGHEST)
    hs1_ref[...] = h * dinv2
    dinv_ref[...] = dinv2


def _d_body(accp_ref, hs1_ref, dinv_ref, b1_ref, w2_ref, wm_ref, z_ref):
    dinv2 = dinv_ref[...]
    a = accp_ref[0] + accp_ref[1] + hs1_ref[...]
    h1 = a * dinv2 + b1_ref[...][None, :]
    h1r = jnp.maximum(h1, 0.0)
    wm2 = jnp.dot(_r16(w2_ref[...]), _r16(wm_ref[...]),
                  preferred_element_type=jnp.float32,
                  precision=lax.Precision.HIGHEST)
    zz = jnp.dot(_r16(h1r), wm2, preferred_element_type=jnp.float32,
                 precision=lax.Precision.HIGHEST) * dinv2
    lane = lax.broadcasted_iota(jnp.int32, (RB, 16), 1)
    z_ref[...] = jnp.where(lane == 0, jnp.broadcast_to(zz, (RB, 16)), 0.0)


def _f_body(g, z_ref, szp_ref, dinv_ref, batch_ref, b2_ref, wm_ref, bm_ref,
            out_ref, sums_scr, cnts_scr):
    i = pl.program_id(0)
    c22 = jnp.dot(b2_ref[...][None, :], wm_ref[...],
                  preferred_element_type=jnp.float32, precision=lax.Precision.HIGHEST)
    szs = szp_ref[0] + szp_ref[1]
    sz2 = jnp.sum(szs, axis=1, keepdims=True)
    zb2 = jnp.sum(z_ref[...], axis=1, keepdims=True)
    v2 = dinv_ref[...] * (sz2 + zb2) + c22
    onehot = (batch_ref[...] ==
              lax.broadcasted_iota(jnp.int32, (RB, g), 1)).astype(jnp.float32)
    ps = jnp.sum(onehot * v2, axis=0, keepdims=True)
    pc = jnp.sum(onehot, axis=0, keepdims=True)

    @pl.when(i == 0)
    def _init():
        sums_scr[...] = ps
        cnts_scr[...] = pc

    @pl.when(i > 0)
    def _accum():
        sums_scr[...] = sums_scr[...] + ps
        cnts_scr[...] = cnts_scr[...] + pc

    @pl.when(i == pl.num_programs(0) - 1)
    def _fin():
        pooled = sums_scr[...] / jnp.maximum(cnts_scr[...], 1.0)
        out_ref[...] = pooled + bm_ref[...][None, :]


def kernel(x, edge_index, batch, W1, b1, W2, b2, Wm, bm):
    n, f = x.shape
    h = W1.shape[1]
    e = edge_index.shape[1]
    g = 64

    npad = ((n + 2047) // 2048) * 2048
    ep = ((e + 32 * EBA - 1) // (32 * EBA)) * 32 * EBA
    nb = npad // RB

    src = edge_index[0]
    dst = edge_index[1]
    pad_idx = (n + jnp.arange(ep - e, dtype=jnp.int32) % (npad - n)).astype(
        jnp.int32)
    src_p = jnp.concatenate([src.astype(jnp.int32), pad_idx])
    dst_p = jnp.concatenate([dst.astype(jnp.int32), pad_idx])
    x_p = jnp.pad(x, ((0, npad - n), (0, 0)))
    batch_p = jnp.pad(batch.astype(jnp.int32), (0, npad - n),
                      constant_values=g)[:, None]
    zeros16 = jnp.zeros((npad, 16), jnp.float32)
    zeros_f = jnp.zeros((npad, f), jnp.float32)
    e0rows = jnp.zeros((EBA, 16), jnp.float32).at[:, 0].set(1.0)

    degp = _deg_kernel(npad, ep)(dst_p, zeros16, e0rows)

    hs1, dinv2d = pl.pallas_call(
        _b_body,
        grid=(nb,),
        in_specs=[
            pl.BlockSpec((RB, f), lambda i: (i, 0)),
            pl.BlockSpec((f, h), lambda i: (0, 0)),
            pl.BlockSpec((NC, RB, 16), lambda i: (0, i, 0)),
        ],
        out_specs=[
            pl.BlockSpec((RB, h), lambda i: (i, 0)),
            pl.BlockSpec((RB, 1), lambda i: (i, 0)),
        ],
        out_shape=[
            jax.ShapeDtypeStruct((npad, h), jnp.float32),
            jax.ShapeDtypeStruct((npad, 1), jnp.float32),
        ],
    )(x_p, W1, degp)

    accp = _row_scatter_kernel(npad, ep, h)(hs1, src_p, dst_p, zeros_f)

    z16 = pl.pallas_call(
        _d_body,
        grid=(nb,),
        in_specs=[
            pl.BlockSpec((NC, RB, h), lambda i: (0, i, 0)),
            pl.BlockSpec((RB, h), lambda i: (i, 0)),
            pl.BlockSpec((RB, 1), lambda i: (i, 0)),
            pl.BlockSpec((h,), lambda i: (0,)),
            pl.BlockSpec((h, h), lambda i: (0, 0)),
            pl.BlockSpec((h, 1), lambda i: (0, 0)),
        ],
        out_specs=pl.BlockSpec((RB, 16), lambda i: (i, 0)),
        out_shape=jax.ShapeDtypeStruct((npad, 16), jnp.float32),
    )(accp, hs1, dinv2d, b1, W2, Wm)

    szp = _z_scatter_kernel(npad, ep)(z16, src_p, dst_p, zeros16)

    out = pl.pallas_call(
        functools.partial(_f_body, g),
        grid=(nb,),
        in_specs=[
            pl.BlockSpec((RB, 16), lambda i: (i, 0)),
            pl.BlockSpec((NC, RB, 16), lambda i: (0, i, 0)),
            pl.BlockSpec((RB, 1), lambda i: (i, 0)),
            pl.BlockSpec((RB, 1), lambda i: (i, 0)),
            pl.BlockSpec((h,), lambda i: (0,)),
            pl.BlockSpec((h, 1), lambda i: (0, 0)),
            pl.BlockSpec((1,), lambda i: (0,)),
        ],
        out_specs=pl.BlockSpec((1, g), lambda i: (0, 0)),
        out_shape=jax.ShapeDtypeStruct((1, g), jnp.float32),
        scratch_shapes=[
            pltpu.VMEM((1, g), jnp.float32),
            pltpu.VMEM((1, g), jnp.float32),
        ],
        compiler_params=pltpu.CompilerParams(
            dimension_semantics=("arbitrary",)),
    )(z16, szp, dinv2d, batch_p, b2, Wm, bm)

    return out.reshape(g, 1)
